# baseline (speedup 1.0000x reference)
import dataclasses
import functools

import jax
import jax.numpy as jnp
from jax import lax
from jax.experimental import pallas as pl
from jax.experimental.pallas import tpu as pltpu
from jax.experimental.pallas import tpu_sc as plsc

_NUM_OPTIONS = 16
_NUM_ACTIONS = 32
_D_SMALL = _NUM_OPTIONS
_D_IOP = _NUM_OPTIONS * _NUM_ACTIONS
_NC, _NS = 2, 16
_NW = _NC * _NS
_L = 16
_LANES = 128
_RPW = _LANES // _D_SMALL
_KIOP = _D_IOP // _LANES

_mesh = plsc.VectorSubcoreMesh(core_axis_name="c", subcore_axis_name="s")


@functools.lru_cache(maxsize=None)
def _build(B, V):
    bpw = B // _NW

    wide_pw = bpw * _D_SMALL // _LANES

    def body(obs_hbm, beta_hbm, iop_hbm, poo_hbm, q_hbm,
             beta_o, iop_o, poo_o, q_o,
             idx_v, j_v, idx8_v,
             iop_b0, iop_b1, beta_s, poo_s, q_s,
             beta_c, poo_c, q_c,
             sem_s, sem_w, sem_a, sem_b, sem_wa, sem_wb):
        wid = lax.axis_index("s") * _NC + lax.axis_index("c")
        base = wid * bpw
        pltpu.sync_copy(obs_hbm.at[pl.ds(base, bpw)], idx_v)

        for m in range(bpw // _L):
            sl = pl.ds(m * _L, _L)
            idx8_v[sl] = lax.shift_right_logical(idx_v[sl], 3)

        c_beta = pltpu.async_copy(beta_hbm.at[idx8_v], beta_s, sem_s)
        c_poo = pltpu.async_copy(poo_hbm.at[idx8_v], poo_s, sem_s)
        c_q = pltpu.async_copy(q_hbm.at[idx8_v], q_s, sem_s)

        lane = lax.iota(jnp.int32, _L)
        rep = lax.shift_right_logical(lane, 2)
        kmod = lax.bitwise_and(lane, 3)
        for c in range(_KIOP):
            for m in range(_LANES // _L):
                ob = plsc.load_gather(idx_v, [rep + (32 * c + 4 * m)])
                j_v[c, pl.ds(m * _L, _L)] = ob * 4 + kmod

        bufs = (iop_b0, iop_b1)
        gsems = (sem_a, sem_b)
        wsems = (sem_wa, sem_wb)

        def fire_gather(c):
            return pltpu.async_copy(
                iop_hbm.at[j_v.at[c]], bufs[c % 2], gsems[c % 2])

        def fire_write(c):
            return pltpu.async_copy(
                bufs[c % 2],
                iop_o.at[pl.ds(_KIOP * base + c * _LANES, _LANES)],
                wsems[c % 2],
            )

        g0 = fire_gather(0)
        g1 = fire_gather(1)

        c_beta.wait()
        c_poo.wait()
        c_q.wait()

        @pl.loop(0, bpw)
        def _(i):
            r = lax.shift_right_logical(i, 3)
            c0 = lax.bitwise_and(i, 7) * _D_SMALL
            i_b = jnp.full((_L,), i, jnp.int32)
            ob = plsc.load_gather(idx_v, [i_b])
            off_v = lax.bitwise_and(ob, 7) * _D_SMALL + lane
            b = plsc.load_gather(beta_s, [i_b, off_v])
            beta_c[r, pl.ds(c0, _D_SMALL)] = 1.0 / (1.0 + jnp.exp(-b))
            poo_c[r, pl.ds(c0, _D_SMALL)] = plsc.load_gather(poo_s, [i_b, off_v])
            q_c[r, pl.ds(c0, _D_SMALL)] = plsc.load_gather(q_s, [i_b, off_v])

        wbase = wid * wide_pw
        w1 = pltpu.async_copy(beta_c, beta_o.at[pl.ds(wbase, wide_pw)], sem_w)
        w2 = pltpu.async_copy(poo_c, poo_o.at[pl.ds(wbase, wide_pw)], sem_w)
        w3 = pltpu.async_copy(q_c, q_o.at[pl.ds(wbase, wide_pw)], sem_w)

        g0.wait()
        w_0 = fire_write(0)
        g1.wait()
        w_1 = fire_write(1)
        w_0.wait()
        g2 = fire_gather(2)
        w_1.wait()
        g3 = fire_gather(3)
        g2.wait()
        w_2 = fire_write(2)
        g3.wait()
        w_3 = fire_write(3)
        w1.wait()
        w2.wait()
        w3.wait()
        w_2.wait()
        w_3.wait()

    cp = pltpu.CompilerParams()
    if "needs_layout_passes" in pltpu.CompilerParams.__dataclass_fields__:
        cp = dataclasses.replace(cp, needs_layout_passes=False)

    return pl.kernel(
        body,
        compiler_params=cp,
        out_type=[
            jax.ShapeDtypeStruct((B * _D_SMALL // _LANES, _LANES), jnp.float32),
            jax.ShapeDtypeStruct((B * _KIOP, _LANES), jnp.float32),
            jax.ShapeDtypeStruct((B * _D_SMALL // _LANES, _LANES), jnp.float32),
            jax.ShapeDtypeStruct((B * _D_SMALL // _LANES, _LANES), jnp.float32),
        ],
        mesh=_mesh,
        scratch_types=[
            pltpu.VMEM((bpw,), jnp.int32),
            pltpu.VMEM((_KIOP, _LANES), jnp.int32),
            pltpu.VMEM((bpw,), jnp.int32),
            pltpu.VMEM((bpw, _LANES), jnp.float32),
            pltpu.VMEM((bpw, _LANES), jnp.float32),
            pltpu.VMEM((bpw, _LANES), jnp.float32),
            pltpu.VMEM((bpw, _LANES), jnp.float32),
            pltpu.VMEM((bpw, _LANES), jnp.float32),
            pltpu.VMEM((bpw * _D_SMALL // _LANES, _LANES), jnp.float32),
            pltpu.VMEM((bpw * _D_SMALL // _LANES, _LANES), jnp.float32),
            pltpu.VMEM((bpw * _D_SMALL // _LANES, _LANES), jnp.float32),
            pltpu.SemaphoreType.DMA,
            pltpu.SemaphoreType.DMA,
            pltpu.SemaphoreType.DMA,
            pltpu.SemaphoreType.DMA,
            pltpu.SemaphoreType.DMA,
            pltpu.SemaphoreType.DMA,
        ],
    )


@jax.jit
def kernel(obs, beta, iop, poo, q):
    B = obs.shape[0]
    V = iop.shape[0]
    beta8 = beta.reshape(V // _RPW, _LANES)
    poo8 = poo.reshape(V // _RPW, _LANES)
    q8 = q.reshape(V // _RPW, _LANES)
    iop4 = iop.reshape(V * _KIOP, _LANES)
    beta_o, iop_o, poo_o, q_o = _build(B, V)(obs, beta8, iop4, poo8, q8)
    return (
        beta_o.reshape(-1),
        iop_o.reshape(-1, _NUM_ACTIONS),
        poo_o.reshape(-1),
        q_o.reshape(-1),
    )


# speedup vs baseline: 3.5782x; 3.5782x over previous
import dataclasses
import functools

import jax
import jax.numpy as jnp
from jax import lax
from jax.experimental import pallas as pl
from jax.experimental.pallas import tpu as pltpu
from jax.experimental.pallas import tpu_sc as plsc

_NUM_OPTIONS = 16
_NUM_ACTIONS = 32
_D_SMALL = _NUM_OPTIONS
_D_IOP = _NUM_OPTIONS * _NUM_ACTIONS
_NC, _NS = 2, 16
_NW = _NC * _NS
_L = 16
_CH = 32
_NCHUNK = 4

_mesh = plsc.VectorSubcoreMesh(core_axis_name="c", subcore_axis_name="s")


def _compiler_params(linear):
    cp = pltpu.CompilerParams()
    if linear:
        cp = dataclasses.replace(cp, use_tc_tiling_on_sc=False)
    if "needs_layout_passes" in pltpu.CompilerParams.__dataclass_fields__:
        cp = dataclasses.replace(cp, needs_layout_passes=False)
    return cp


@functools.lru_cache(maxsize=None)
def _build_iop(B, V):
    bpw = B // _NW
    assert bpw == _CH * _NCHUNK

    def body(obs_hbm, iop_hbm, out2,
             idx4, buf0, buf1, stage, sem_a, sem_b, sem_w):
        wid = lax.axis_index("s") * _NC + lax.axis_index("c")
        base = wid * bpw
        for c in range(_NCHUNK):
            pltpu.sync_copy(obs_hbm.at[pl.ds(base + c * _CH, _CH)], idx4.at[c])

        bufs = (buf0, buf1)
        sems = (sem_a, sem_b)

        def fire(c):
            return pltpu.async_copy(
                iop_hbm.at[idx4.at[c]], bufs[c % 2], sems[c % 2])

        lane = lax.iota(jnp.int32, _L)

        def transpose_chunk(c, buf):
            @pl.loop(0, _CH)
            def _(bl):
                col0 = (c * _CH + bl) * _NUM_OPTIONS
                for a in range(_NUM_ACTIONS):
                    bl_b = jnp.full((_L,), bl, jnp.int32)
                    v = plsc.load_gather(buf, [bl_b, lane * _NUM_ACTIONS + a])
                    stage[a, pl.ds(col0, _NUM_OPTIONS)] = v

        g0 = fire(0)
        g1 = fire(1)
        g0.wait()
        transpose_chunk(0, buf0)
        g2 = fire(2)
        g1.wait()
        transpose_chunk(1, buf1)
        g3 = fire(3)
        g2.wait()
        transpose_chunk(2, buf0)
        g3.wait()
        transpose_chunk(3, buf1)
        pltpu.sync_copy(
            stage,
            out2.at[:, pl.ds(wid * (bpw * _NUM_OPTIONS), bpw * _NUM_OPTIONS)],
        )

    return pl.kernel(
        body,
        compiler_params=_compiler_params(linear=False),
        out_type=[
            jax.ShapeDtypeStruct((_NUM_ACTIONS, B * _NUM_OPTIONS), jnp.float32)
        ],
        mesh=_mesh,
        scratch_types=[
            pltpu.VMEM((_NCHUNK, _CH), jnp.int32),
            pltpu.VMEM((_CH, _D_IOP), jnp.float32),
            pltpu.VMEM((_CH, _D_IOP), jnp.float32),
            pltpu.VMEM((_NUM_ACTIONS, bpw * _NUM_OPTIONS), jnp.float32),
            pltpu.SemaphoreType.DMA,
            pltpu.SemaphoreType.DMA,
            pltpu.SemaphoreType.DMA,
        ],
    )


@functools.lru_cache(maxsize=None)
def _build_small(B, V):
    bpw = B // _NW

    def body(obs_hbm, beta_hbm, poo_hbm, q_hbm,
             beta_o, poo_o, q_o,
             idx_v, beta_v, poo_v, q_v, sem):
        wid = lax.axis_index("s") * _NC + lax.axis_index("c")
        base = wid * bpw
        pltpu.sync_copy(obs_hbm.at[pl.ds(base, bpw)], idx_v)
        c1 = pltpu.async_copy(beta_hbm.at[idx_v], beta_v, sem)
        c2 = pltpu.async_copy(poo_hbm.at[idx_v], poo_v, sem)
        c3 = pltpu.async_copy(q_hbm.at[idx_v], q_v, sem)
        c1.wait()

        @pl.loop(0, bpw)
        def _(i):
            row = beta_v[i, :]
            beta_v[i, :] = 1.0 / (1.0 + jnp.exp(-row))

        pltpu.sync_copy(beta_v, beta_o.at[pl.ds(base, bpw)])
        c2.wait()
        pltpu.sync_copy(poo_v, poo_o.at[pl.ds(base, bpw)])
        c3.wait()
        pltpu.sync_copy(q_v, q_o.at[pl.ds(base, bpw)])

    return pl.kernel(
        body,
        compiler_params=_compiler_params(linear=True),
        out_type=[
            jax.ShapeDtypeStruct((B, _D_SMALL), jnp.float32),
            jax.ShapeDtypeStruct((B, _D_SMALL), jnp.float32),
            jax.ShapeDtypeStruct((B, _D_SMALL), jnp.float32),
        ],
        mesh=_mesh,
        scratch_types=[
            pltpu.VMEM((bpw,), jnp.int32),
            pltpu.VMEM((bpw, _D_SMALL), jnp.float32),
            pltpu.VMEM((bpw, _D_SMALL), jnp.float32),
            pltpu.VMEM((bpw, _D_SMALL), jnp.float32),
            pltpu.SemaphoreType.DMA,
        ],
    )


@jax.jit
def kernel(obs, beta, iop, poo, q):
    B = obs.shape[0]
    V = iop.shape[0]
    iop2 = iop.reshape(V, _D_IOP)
    (out2,) = _build_iop(B, V)(obs, iop2)
    beta_o, poo_o, q_o = _build_small(B, V)(obs, beta, poo, q)
    return (
        beta_o.reshape(-1),
        jnp.transpose(out2),
        poo_o.reshape(-1),
        q_o.reshape(-1),
    )


# speedup vs baseline: 4.2098x; 1.1765x over previous
import dataclasses
import functools

import jax
import jax.numpy as jnp
from jax import lax
from jax.experimental import pallas as pl
from jax.experimental.pallas import tpu as pltpu
from jax.experimental.pallas import tpu_sc as plsc

_NUM_OPTIONS = 16
_NUM_ACTIONS = 32
_D_SMALL = _NUM_OPTIONS
_D_IOP = _NUM_OPTIONS * _NUM_ACTIONS
_NC, _NS = 2, 16
_NW = _NC * _NS
_L = 16
_CH = 32
_NCHUNK = 4

_mesh = plsc.VectorSubcoreMesh(core_axis_name="c", subcore_axis_name="s")


def _compiler_params():
    cp = pltpu.CompilerParams()
    if "needs_layout_passes" in pltpu.CompilerParams.__dataclass_fields__:
        cp = dataclasses.replace(cp, needs_layout_passes=False)
    return cp


@functools.lru_cache(maxsize=None)
def _build_iop(B, V):
    bpw = B // _NW
    assert bpw == _CH * _NCHUNK

    def body(obs_hbm, iop_hbm, out2,
             idx4, buf0, buf1, stage, sem_a, sem_b, sem_w):
        wid = lax.axis_index("s") * _NC + lax.axis_index("c")
        base = wid * bpw
        for c in range(_NCHUNK):
            pltpu.sync_copy(obs_hbm.at[pl.ds(base + c * _CH, _CH)], idx4.at[c])

        bufs = (buf0, buf1)
        sems = (sem_a, sem_b)

        def fire(c):
            return pltpu.async_copy(
                iop_hbm.at[idx4.at[c]], bufs[c % 2], sems[c % 2])

        lane = lax.iota(jnp.int32, _L)

        def transpose_chunk(c, buf):
            @pl.loop(0, _CH)
            def _(bl):
                col0 = (c * _CH + bl) * _NUM_OPTIONS
                for a in range(_NUM_ACTIONS):
                    bl_b = jnp.full((_L,), bl, jnp.int32)
                    v = plsc.load_gather(buf, [bl_b, lane * _NUM_ACTIONS + a])
                    stage[a, pl.ds(col0, _NUM_OPTIONS)] = v

        g0 = fire(0)
        g1 = fire(1)
        g0.wait()
        transpose_chunk(0, buf0)
        g2 = fire(2)
        g1.wait()
        transpose_chunk(1, buf1)
        g3 = fire(3)
        g2.wait()
        transpose_chunk(2, buf0)
        g3.wait()
        transpose_chunk(3, buf1)
        pltpu.sync_copy(
            stage,
            out2.at[:, pl.ds(wid * (bpw * _NUM_OPTIONS), bpw * _NUM_OPTIONS)],
        )

    return pl.kernel(
        body,
        compiler_params=_compiler_params(),
        out_type=[
            jax.ShapeDtypeStruct((_NUM_ACTIONS, B * _NUM_OPTIONS), jnp.float32)
        ],
        mesh=_mesh,
        scratch_types=[
            pltpu.VMEM((_NCHUNK, _CH), jnp.int32),
            pltpu.VMEM((_CH, _D_IOP), jnp.float32),
            pltpu.VMEM((_CH, _D_IOP), jnp.float32),
            pltpu.VMEM((_NUM_ACTIONS, (B // _NW) * _NUM_OPTIONS), jnp.float32),
            pltpu.SemaphoreType.DMA,
            pltpu.SemaphoreType.DMA,
            pltpu.SemaphoreType.DMA,
        ],
    )


@functools.lru_cache(maxsize=None)
def _build_small(B, V):
    bpw = B // _NW
    wide_pw = bpw * _D_SMALL // 128

    def body(obs_hbm, betaT, pooT, qT,
             beta_o, poo_o, q_o,
             idx_v, sb0, sb1, beta_c, poo_c, q_c,
             sem_a, sem_b):
        wid = lax.axis_index("s") * _NC + lax.axis_index("c")
        base = wid * bpw
        pltpu.sync_copy(obs_hbm.at[pl.ds(base, bpw)], idx_v)

        sbufs = (sb0, sb1)
        sems = (sem_a, sem_b)
        lane = lax.iota(jnp.int32, _L)

        def fire(o, par):
            cb = pl.multiple_of(lax.bitwise_and(o, jnp.int32(~127)), 128)
            sb = sbufs[par]
            pltpu.async_copy(betaT.at[:, pl.ds(cb, 128)], sb.at[0], sems[par])
            pltpu.async_copy(pooT.at[:, pl.ds(cb, 128)], sb.at[1], sems[par])
            pltpu.async_copy(qT.at[:, pl.ds(cb, 128)], sb.at[2], sems[par])

        def drain(par):
            for t in range(3):
                pltpu.make_async_copy(
                    betaT.at[:, pl.ds(0, 128)], sbufs[par].at[t],
                    sems[par]).wait()

        def process(o, i, par):
            om = lax.bitwise_and(o, 127)
            r8 = lax.shift_right_logical(i, 3)
            c0 = lax.bitwise_and(i, 7) * _D_SMALL
            sb = sbufs[par]
            om_b = jnp.full((_L,), om, jnp.int32)
            bv = plsc.load_gather(sb.at[0], [lane, om_b])
            beta_c[r8, pl.ds(c0, _D_SMALL)] = 1.0 / (1.0 + jnp.exp(-bv))
            poo_c[r8, pl.ds(c0, _D_SMALL)] = plsc.load_gather(
                sb.at[1], [lane, om_b])
            q_c[r8, pl.ds(c0, _D_SMALL)] = plsc.load_gather(
                sb.at[2], [lane, om_b])

        ov0 = idx_v[pl.ds(0, _L)]
        fire(ov0[0], 0)
        fire(ov0[1], 1)

        @pl.loop(0, bpw // _L)
        def _(m):
            ov = idx_v[pl.ds(m * _L, _L)]
            nxt = jnp.minimum((m + 1) * _L, bpw - _L)
            ovn = idx_v[pl.ds(nxt, _L)]
            last = m == bpw // _L - 1
            for l in range(_L):
                p = l % 2
                drain(p)
                process(ov[l], m * _L + l, p)
                if l < _L - 2:
                    fire(ov[l + 2], p)
                else:
                    @pl.when(jnp.logical_not(last))
                    def _():
                        fire(ovn[l - (_L - 2)], p)

        pltpu.sync_copy(beta_c, beta_o.at[pl.ds(wid * wide_pw, wide_pw)])
        pltpu.sync_copy(poo_c, poo_o.at[pl.ds(wid * wide_pw, wide_pw)])
        pltpu.sync_copy(q_c, q_o.at[pl.ds(wid * wide_pw, wide_pw)])

    return pl.kernel(
        body,
        compiler_params=_compiler_params(),
        out_type=[
            jax.ShapeDtypeStruct((B * _D_SMALL // 128, 128), jnp.float32),
            jax.ShapeDtypeStruct((B * _D_SMALL // 128, 128), jnp.float32),
            jax.ShapeDtypeStruct((B * _D_SMALL // 128, 128), jnp.float32),
        ],
        mesh=_mesh,
        scratch_types=[
            pltpu.VMEM((bpw,), jnp.int32),
            pltpu.VMEM((3, _D_SMALL, 128), jnp.float32),
            pltpu.VMEM((3, _D_SMALL, 128), jnp.float32),
            pltpu.VMEM((bpw * _D_SMALL // 128, 128), jnp.float32),
            pltpu.VMEM((bpw * _D_SMALL // 128, 128), jnp.float32),
            pltpu.VMEM((bpw * _D_SMALL // 128, 128), jnp.float32),
            pltpu.SemaphoreType.DMA,
            pltpu.SemaphoreType.DMA,
        ],
    )


@jax.jit
def kernel(obs, beta, iop, poo, q):
    B = obs.shape[0]
    V = iop.shape[0]
    iop2 = iop.reshape(V, _D_IOP)
    (out2,) = _build_iop(B, V)(obs, iop2)
    beta_o, poo_o, q_o = _build_small(B, V)(obs, beta.T, poo.T, q.T)
    return (
        beta_o.reshape(-1),
        jnp.transpose(out2),
        poo_o.reshape(-1),
        q_o.reshape(-1),
    )
